# baseline (speedup 1.0000x reference)
import jax
import jax.numpy as jnp
from jax import lax
from jax.experimental import pallas as pl
from jax.experimental.pallas import tpu as pltpu, tpu_sc as plsc

N = 4096
D = 128
BLK = 256
K_RIPS = 32
K_KDE = 16
E = N * K_RIPS
CHUNK = 8192
CAP = 32768
DESTNUM = 10


def _k1_body(xb_ref, x_ref, f_ref, nbrs_ref):
    blk = pl.program_id(0)
    xb = xb_ref[...]
    xall = x_ref[...]
    sb = jnp.sum(xb * xb, axis=1)
    sa = jnp.sum(xall * xall, axis=1)
    g = lax.dot_general(xb, xall, (((1,), (1,)), ((), ())),
                        preferred_element_type=jnp.float32)
    d2 = jnp.maximum(sb[:, None] + sa[None, :] - 2.0 * g, 0.0)
    col = lax.broadcasted_iota(jnp.int32, (BLK, N), 1)
    rowg = lax.broadcasted_iota(jnp.int32, (BLK, N), 0) + blk * BLK
    neg = -d2 - jnp.where(col == rowg, 1e9, 0.0).astype(jnp.float32)
    colk = lax.broadcasted_iota(jnp.int32, (BLK, K_RIPS), 1)

    def extract(k, carry):
        neg, acc, idxacc = carry
        m = jnp.max(neg, axis=1)
        am = jnp.min(jnp.where(neg == m[:, None], col, N), axis=1)
        acc = acc + jnp.where(k < K_KDE, jnp.exp(m), 0.0)
        idxacc = jnp.where(colk == k, am[:, None], idxacc)
        neg = jnp.where(col == am[:, None], -jnp.inf, neg)
        return neg, acc, idxacc

    _, acc, idxacc = lax.fori_loop(
        0, K_RIPS, extract,
        (neg, jnp.zeros((BLK,), jnp.float32),
         jnp.zeros((BLK, K_RIPS), jnp.int32)))
    f_ref[...] = acc * (1.0 / K_KDE)
    nbrs_ref[...] = idxacc


def _density_topk(x):
    return pl.pallas_call(
        _k1_body,
        grid=(N // BLK,),
        in_specs=[pl.BlockSpec((BLK, D), lambda i: (i, 0)),
                  pl.BlockSpec((N, D), lambda i: (0, 0))],
        out_specs=[pl.BlockSpec((BLK,), lambda i: (i,)),
                   pl.BlockSpec((BLK, K_RIPS), lambda i: (i, 0))],
        out_shape=[jax.ShapeDtypeStruct((N,), jnp.float32),
                   jax.ShapeDtypeStruct((N, K_RIPS), jnp.int32)],
    )(x, x)


def _k2_body(f_ref, rank_ref):
    fall = f_ref[...]
    colu = lax.broadcasted_iota(jnp.int32, (BLK, N), 1)
    rowi = lax.broadcasted_iota(jnp.int32, (BLK, N), 0)

    def chunk(i, _):
        fb = f_ref[pl.ds(i * BLK, BLK)]
        vglob = rowi + i * BLK
        gt = fall[None, :] > fb[:, None]
        tie = (fall[None, :] == fb[:, None]) & (colu < vglob)
        r = jnp.sum(jnp.where(gt | tie, 1, 0).astype(jnp.int32), axis=1)
        rank_ref[pl.ds(i * BLK, BLK)] = r
        return 0

    lax.fori_loop(0, N // BLK, chunk, 0)


def _ranks(f):
    return pl.pallas_call(
        _k2_body,
        out_shape=jax.ShapeDtypeStruct((N,), jnp.int32),
    )(f)


def _k3_body(nbrs_hbm, rank_hbm, f_hbm, codes_hbm, fsorted_hbm,
             rank_v, f_v, fs_v, nbuf, cbuf, sem):
    wid = lax.axis_index("s") * 2 + lax.axis_index("c")

    @pl.when(wid == 0)
    def _():
        pltpu.sync_copy(rank_hbm, rank_v)
        pltpu.sync_copy(f_hbm, f_v)

        def scat(i, _):
            idx = rank_v[pl.ds(i * 16, 16)]
            fv = f_v[pl.ds(i * 16, 16)]
            plsc.store_scatter(fs_v, [idx], fv)
            return 0

        lax.fori_loop(0, N // 16, scat, 0)
        pltpu.sync_copy(fs_v, fsorted_hbm)

        def chunk_body(ch, _):
            pltpu.sync_copy(nbrs_hbm.at[pl.ds(ch * CHUNK, CHUNK)], nbuf)

            def vec_body(j, _):
                base = ch * CHUNK + j * 16
                flat = lax.iota(jnp.int32, 16) + base
                u = lax.shift_right_logical(flat, 5)
                a = plsc.load_gather(rank_v, [u])
                w = nbuf[pl.ds(j * 16, 16)]
                b = plsc.load_gather(rank_v, [w])
                code = jnp.bitwise_or(
                    lax.shift_left(jnp.maximum(a, b), 12), jnp.minimum(a, b))
                cbuf[pl.ds(j * 16, 16)] = code
                return 0

            lax.fori_loop(0, CHUNK // 16, vec_body, 0)
            pltpu.sync_copy(cbuf, codes_hbm.at[pl.ds(ch * CHUNK, CHUNK)])
            return 0

        lax.fori_loop(0, E // CHUNK, chunk_body, 0)


_SC_PARAMS = pltpu.CompilerParams(use_tc_tiling_on_sc=False,
                                  needs_layout_passes=False)


def _edge_codes(nbrs_flat, rank, f):
    mesh = plsc.VectorSubcoreMesh(core_axis_name="c", subcore_axis_name="s")
    kfn = pl.kernel(
        _k3_body,
        mesh=mesh,
        compiler_params=_SC_PARAMS,
        out_type=[jax.ShapeDtypeStruct((E,), jnp.int32),
                  jax.ShapeDtypeStruct((N,), jnp.float32)],
        scratch_types=[pltpu.VMEM((N,), jnp.int32),
                       pltpu.VMEM((N,), jnp.float32),
                       pltpu.VMEM((N,), jnp.float32),
                       pltpu.VMEM((CHUNK,), jnp.int32),
                       pltpu.VMEM((CHUNK,), jnp.int32),
                       pltpu.SemaphoreType.DMA],
    )
    return kfn(nbrs_flat, rank, f)


def _sload(ref, i):
    return plsc.load_gather(ref, [jnp.full((16,), i, jnp.int32)])[0]


def _sstore(ref, i, v):
    plsc.store_scatter(ref, [jnp.full((16,), i, jnp.int32)],
                       jnp.full((16,), v),
                       mask=lax.iota(jnp.int32, 16) == 0)


def _k4_body(codes_hbm, fsorted_hbm, out_hbm,
             fs_v, parent, pers, cur, win, cbuf, obuf, sem):
    wid = lax.axis_index("s") * 2 + lax.axis_index("c")

    @pl.when(wid == 0)
    def _():
        pltpu.sync_copy(fsorted_hbm, fs_v)

        def init(i, _):
            parent[pl.ds(i * 16, 16)] = lax.iota(jnp.int32, 16) + i * 16
            pers[pl.ds(i * 16, 16)] = jnp.full((16,), -1.0, jnp.float32)
            cur[pl.ds(i * 16, 16)] = jnp.zeros((16,), jnp.int32)
            return 0

        lax.fori_loop(0, N // 16, init, 0)

        def hist_chunk(ch, _):
            pltpu.sync_copy(codes_hbm.at[pl.ds(ch * CHUNK, CHUNK)], cbuf)

            def hist_v(j, _):
                kv = lax.shift_right_logical(cbuf[pl.ds(j * 16, 16)], 12)
                for lane in range(16):
                    k = kv[lane]
                    _sstore(cur, k, _sload(cur, k) + 1)
                return 0

            lax.fori_loop(0, CHUNK // 16, hist_v, 0)
            return 0

        lax.fori_loop(0, E // CHUNK, hist_chunk, 0)

        def pfx(j, run):
            v = cur[pl.ds(j * 16, 16)]
            cs = jnp.cumsum(v)
            excl = cs - v + run
            cur[pl.ds(j * 16, 16)] = excl
            return run + cs[15]

        lax.fori_loop(0, N // 16, pfx, jnp.int32(0))

        def nxt_cum(h):
            return jnp.where(h + 1 < N,
                             _sload(cur, jnp.minimum(h + 1, N - 1)),
                             jnp.int32(E))

        def seg_cond(lo):
            return lo < N

        def seg_body(lo):
            base = _sload(cur, lo)

            def hi_cond(h):
                return jnp.logical_and(h < N, nxt_cum(h) - base <= CAP)

            hi = lax.while_loop(hi_cond, lambda h: h + 1, lo)
            seg_cnt = jnp.where(hi < N,
                                _sload(cur, jnp.minimum(hi, N - 1)),
                                jnp.int32(E)) - base

            def fill_chunk(ch, _):
                pltpu.sync_copy(codes_hbm.at[pl.ds(ch * CHUNK, CHUNK)], cbuf)

                def fill_v(j, _):
                    cv = cbuf[pl.ds(j * 16, 16)]
                    kv = lax.shift_right_logical(cv, 12)
                    for lane in range(16):
                        k = kv[lane]

                        @pl.when(jnp.logical_and(k >= lo, k < hi))
                        def _():
                            pos = _sload(cur, k)
                            _sstore(cur, k, pos + 1)
                            _sstore(win, pos - base, cv[lane])

                    return 0

                lax.fori_loop(0, CHUNK // 16, fill_v, 0)
                return 0

            lax.fori_loop(0, E // CHUNK, fill_chunk, 0)

            def scan_e(e, carry):
                cur_key, rk = carry
                c = _sload(win, e)
                k = lax.shift_right_logical(c, 12)
                o = jnp.bitwise_and(c, 4095)
                rk = jnp.where(k != cur_key, k, rk)

                def find_cond(r):
                    return _sload(parent, r) != r

                ro = lax.while_loop(find_cond, lambda r: _sload(parent, r), o)
                _sstore(parent, o, ro)

                dying = jnp.maximum(rk, ro)
                main = jnp.minimum(rk, ro)

                @pl.when(ro != rk)
                def _():
                    _sstore(parent, dying, main)

                    @pl.when(dying != k)
                    def _():
                        _sstore(pers, dying, _sload(fs_v, dying) - _sload(fs_v, k))

                return (k, jnp.where(ro != rk, main, rk))

            lax.fori_loop(0, seg_cnt, scan_e,
                          (jnp.int32(-1), jnp.int32(-1)))
            return hi

        lax.while_loop(seg_cond, seg_body, jnp.int32(0))

        fmin = _sload(fs_v, N - 1)

        def fin(i, carry):
            sumsq, kcnt = carry
            base = i * 16
            rvec = lax.iota(jnp.int32, 16) + base
            par = parent[pl.ds(base, 16)]
            death = pers[pl.ds(base, 16)]
            fsv = fs_v[pl.ds(base, 16)]
            is_root = par == rvec
            valid = jnp.logical_or(is_root, death >= 0.0)
            val = jnp.where(is_root, fsv - fmin, death)
            val = jnp.where(valid, val, jnp.float32(-1.0))
            pers[pl.ds(base, 16)] = val
            sumsq = sumsq + jnp.sum(jnp.where(valid, val * val, 0.0))
            kcnt = kcnt + jnp.sum(jnp.where(valid, 1, 0).astype(jnp.int32))
            return (sumsq, kcnt)

        sumsq, kcnt = lax.fori_loop(0, N // 16, fin,
                                    (jnp.float32(0.0), jnp.int32(0)))

        def top_j(j, carry):
            loss = carry

            def mx(i, m):
                return jnp.maximum(m, jnp.max(pers[pl.ds(i * 16, 16)]))

            m = lax.fori_loop(0, N // 16, mx, jnp.float32(-3.0))

            def locate(i, st):
                found, pos = st
                chunk = pers[pl.ds(i * 16, 16)]
                hit = jnp.logical_and(jnp.logical_not(found),
                                      jnp.max(chunk) >= m)
                lane = jnp.max(plsc.all_reduce_ffs(chunk == m))
                return (jnp.logical_or(found, hit),
                        jnp.where(hit, i * 16 + lane, pos))

            _, pos = lax.fori_loop(0, N // 16, locate,
                                   (jnp.bool_(False), jnp.int32(0)))
            _sstore(pers, pos, jnp.float32(-3.0))
            return loss + jnp.where(j < jnp.minimum(kcnt, DESTNUM),
                                    1.0 - 2.0 * m, 0.0)

        loss = lax.fori_loop(0, DESTNUM, top_j, sumsq)
        obuf[...] = jnp.full((16,), 0.0, jnp.float32) + loss
        pltpu.sync_copy(obuf, out_hbm)


def _persistence_loss(codes, fsorted):
    mesh = plsc.VectorSubcoreMesh(core_axis_name="c", subcore_axis_name="s")
    kfn = pl.kernel(
        _k4_body,
        mesh=mesh,
        compiler_params=_SC_PARAMS,
        out_type=jax.ShapeDtypeStruct((16,), jnp.float32),
        scratch_types=[pltpu.VMEM((N,), jnp.float32),
                       pltpu.VMEM((N,), jnp.int32),
                       pltpu.VMEM((N,), jnp.float32),
                       pltpu.VMEM((N,), jnp.int32),
                       pltpu.VMEM((CAP,), jnp.int32),
                       pltpu.VMEM((CHUNK,), jnp.int32),
                       pltpu.VMEM((16,), jnp.float32),
                       pltpu.SemaphoreType.DMA],
    )
    return kfn(codes, fsorted)


def kernel(x):
    f, nbrs = _density_topk(x)
    rank = _ranks(f)
    codes, fsorted = _edge_codes(nbrs.reshape(E), rank, f)
    out = _persistence_loss(codes, fsorted)
    return out[0]


# speedup vs baseline: 2.6583x; 2.6583x over previous
import jax
import jax.numpy as jnp
from jax import lax
from jax.experimental import pallas as pl
from jax.experimental.pallas import tpu as pltpu, tpu_sc as plsc

N = 4096
D = 128
BLK = 256
K_RIPS = 32
K_KDE = 16
E = N * K_RIPS
CHUNK = 8192
NSUB = 16
TGT = E // NSUB
MAXDEG = N - 1 + K_RIPS
CAPW = 12320
DESTNUM = 10


def _k1_body(xb_ref, x_ref, f_ref, nbrs_ref):
    blk = pl.program_id(0)
    xb = xb_ref[...]
    xall = x_ref[...]
    sb = jnp.sum(xb * xb, axis=1)
    sa = jnp.sum(xall * xall, axis=1)
    g = lax.dot_general(xb, xall, (((1,), (1,)), ((), ())),
                        preferred_element_type=jnp.float32)
    d2 = jnp.maximum(sb[:, None] + sa[None, :] - 2.0 * g, 0.0)
    col = lax.broadcasted_iota(jnp.int32, (BLK, N), 1)
    rowg = lax.broadcasted_iota(jnp.int32, (BLK, N), 0) + blk * BLK
    neg = -d2 - jnp.where(col == rowg, 1e9, 0.0).astype(jnp.float32)
    colk = lax.broadcasted_iota(jnp.int32, (BLK, K_RIPS), 1)

    def extract(k, carry):
        neg, acc, idxacc = carry
        m = jnp.max(neg, axis=1)
        am = jnp.min(jnp.where(neg == m[:, None], col, N), axis=1)
        acc = acc + jnp.where(k < K_KDE, jnp.exp(m), 0.0)
        idxacc = jnp.where(colk == k, am[:, None], idxacc)
        neg = jnp.where(col == am[:, None], -jnp.inf, neg)
        return neg, acc, idxacc

    _, acc, idxacc = lax.fori_loop(
        0, K_RIPS, extract,
        (neg, jnp.zeros((BLK,), jnp.float32),
         jnp.zeros((BLK, K_RIPS), jnp.int32)))
    f_ref[...] = acc * (1.0 / K_KDE)
    nbrs_ref[...] = idxacc


def _density_topk(x):
    return pl.pallas_call(
        _k1_body,
        grid=(N // BLK,),
        in_specs=[pl.BlockSpec((BLK, D), lambda i: (i, 0)),
                  pl.BlockSpec((N, D), lambda i: (0, 0))],
        out_specs=[pl.BlockSpec((BLK,), lambda i: (i,)),
                   pl.BlockSpec((BLK, K_RIPS), lambda i: (i, 0))],
        out_shape=[jax.ShapeDtypeStruct((N,), jnp.float32),
                   jax.ShapeDtypeStruct((N, K_RIPS), jnp.int32)],
    )(x, x)


def _k2_body(f_ref, rank_ref):
    fall = f_ref[...]
    colu = lax.broadcasted_iota(jnp.int32, (BLK, N), 1)
    rowi = lax.broadcasted_iota(jnp.int32, (BLK, N), 0)

    def chunk(i, _):
        fb = f_ref[pl.ds(i * BLK, BLK)]
        vglob = rowi + i * BLK
        gt = fall[None, :] > fb[:, None]
        tie = (fall[None, :] == fb[:, None]) & (colu < vglob)
        r = jnp.sum(jnp.where(gt | tie, 1, 0).astype(jnp.int32), axis=1)
        rank_ref[pl.ds(i * BLK, BLK)] = r
        return 0

    lax.fori_loop(0, N // BLK, chunk, 0)


def _ranks(f):
    return pl.pallas_call(
        _k2_body,
        out_shape=jax.ShapeDtypeStruct((N,), jnp.int32),
    )(f)


def _k3_body(nbrs_hbm, rank_hbm, f_hbm, codes_hbm, fsorted_hbm,
             rank_v, f_v, fs_v, nbuf, cbuf, sem):
    wid = lax.axis_index("s") * 2 + lax.axis_index("c")

    @pl.when(wid == 0)
    def _():
        pltpu.sync_copy(rank_hbm, rank_v)
        pltpu.sync_copy(f_hbm, f_v)

        def scat(i, _):
            idx = rank_v[pl.ds(i * 16, 16)]
            fv = f_v[pl.ds(i * 16, 16)]
            plsc.store_scatter(fs_v, [idx], fv)
            return 0

        lax.fori_loop(0, N // 16, scat, 0)
        pltpu.sync_copy(fs_v, fsorted_hbm)

        def chunk_body(ch, _):
            pltpu.sync_copy(nbrs_hbm.at[pl.ds(ch * CHUNK, CHUNK)], nbuf)

            def vec_body(j, _):
                base = ch * CHUNK + j * 16
                flat = lax.iota(jnp.int32, 16) + base
                u = lax.shift_right_logical(flat, 5)
                a = plsc.load_gather(rank_v, [u])
                w = nbuf[pl.ds(j * 16, 16)]
                b = plsc.load_gather(rank_v, [w])
                code = jnp.bitwise_or(
                    lax.shift_left(jnp.maximum(a, b), 12), jnp.minimum(a, b))
                cbuf[pl.ds(j * 16, 16)] = code
                return 0

            lax.fori_loop(0, CHUNK // 16, vec_body, 0)
            pltpu.sync_copy(cbuf, codes_hbm.at[pl.ds(ch * CHUNK, CHUNK)])
            return 0

        lax.fori_loop(0, E // CHUNK, chunk_body, 0)


_SC_PARAMS = pltpu.CompilerParams(use_tc_tiling_on_sc=False,
                                  needs_layout_passes=False)


def _edge_codes(nbrs_flat, rank, f):
    mesh = plsc.VectorSubcoreMesh(core_axis_name="c", subcore_axis_name="s")
    kfn = pl.kernel(
        _k3_body,
        mesh=mesh,
        compiler_params=_SC_PARAMS,
        out_type=[jax.ShapeDtypeStruct((E,), jnp.int32),
                  jax.ShapeDtypeStruct((N,), jnp.float32)],
        scratch_types=[pltpu.VMEM((N,), jnp.int32),
                       pltpu.VMEM((N,), jnp.float32),
                       pltpu.VMEM((N,), jnp.float32),
                       pltpu.VMEM((CHUNK,), jnp.int32),
                       pltpu.VMEM((CHUNK,), jnp.int32),
                       pltpu.SemaphoreType.DMA],
    )
    return kfn(nbrs_flat, rank, f)


def _sload(ref, i):
    return plsc.load_gather(ref, [jnp.full((16,), i, jnp.int32)])[0]


def _sstore(ref, i, v):
    plsc.store_scatter(ref, [jnp.full((16,), i, jnp.int32)],
                       jnp.full((16,), v),
                       mask=lax.iota(jnp.int32, 16) == 0)


def _k4_body(codes_hbm, fsorted_hbm, out_hbm, hist_hbm, cum_hbm, swin_hbm,
             fs_v, parent, pers, cur, hbuf, win, cbuf, bnd, obuf, sem):
    cid = lax.axis_index("c")
    sid = lax.axis_index("s")

    def _locate(t):
        def cnt(i, acc):
            v = cur[pl.ds(i * 16, 16)]
            return acc + jnp.sum(jnp.where(v < t, 1, 0).astype(jnp.int32))

        return lax.fori_loop(0, N // 16, cnt, jnp.int32(0))

    def _cumval(x):
        return jnp.where(x < N, _sload(cur, jnp.minimum(x, N - 1)),
                         jnp.int32(E))

    @pl.when(cid == 0)
    def _():
        def zero(i, _):
            cur[pl.ds(i * 16, 16)] = jnp.zeros((16,), jnp.int32)
            return 0

        lax.fori_loop(0, N // 16, zero, 0)
        pltpu.sync_copy(codes_hbm.at[pl.ds(sid * CHUNK, CHUNK)], cbuf)

        def hist_v(j, _):
            kv = lax.shift_right_logical(cbuf[pl.ds(j * 16, 16)], 12)
            for lane in range(16):
                k = kv[lane]
                _sstore(cur, k, _sload(cur, k) + 1)
            return 0

        lax.fori_loop(0, CHUNK // 16, hist_v, 0)
        pltpu.sync_copy(cur, hist_hbm.at[sid])
        plsc.subcore_barrier()

        @pl.when(sid == 0)
        def _():
            def addrow(r, _):
                pltpu.sync_copy(hist_hbm.at[r], hbuf)

                def av(i, _):
                    cur[pl.ds(i * 16, 16)] = (cur[pl.ds(i * 16, 16)]
                                              + hbuf[pl.ds(i * 16, 16)])
                    return 0

                lax.fori_loop(0, N // 16, av, 0)
                return 0

            lax.fori_loop(1, NSUB, addrow, 0)

            def pfx(j, run):
                v = cur[pl.ds(j * 16, 16)]
                cs = jnp.cumsum(v)
                cur[pl.ds(j * 16, 16)] = cs - v + run
                return run + cs[15]

            lax.fori_loop(0, N // 16, pfx, jnp.int32(0))
            pltpu.sync_copy(cur, cum_hbm)

        plsc.subcore_barrier()

        pltpu.sync_copy(cum_hbm, cur)
        lo = _locate(sid * TGT)
        hi = _locate((sid + 1) * TGT)
        base = _cumval(lo)

        def fill_chunk(ch, _):
            pltpu.sync_copy(codes_hbm.at[pl.ds(ch * CHUNK, CHUNK)], cbuf)

            def fill_v(j, _):
                cv = cbuf[pl.ds(j * 16, 16)]
                kv = lax.shift_right_logical(cv, 12)
                inr = jnp.logical_and(kv >= lo, kv < hi)
                nhit = plsc.all_reduce_population_count(inr)[0]

                @pl.when(nhit > 0)
                def _():
                    for lane in range(16):
                        k = kv[lane]

                        @pl.when(jnp.logical_and(k >= lo, k < hi))
                        def _():
                            pos = _sload(cur, k)
                            _sstore(cur, k, pos + 1)
                            _sstore(win, pos - base, cv[lane])

                return 0

            lax.fori_loop(0, CHUNK // 16, fill_v, 0)
            return 0

        lax.fori_loop(0, E // CHUNK, fill_chunk, 0)
        pltpu.sync_copy(win, swin_hbm.at[pl.ds(sid * CAPW, CAPW)])
        plsc.subcore_barrier()

        @pl.when(sid == 0)
        def _():
            pltpu.sync_copy(fsorted_hbm, fs_v)
            pltpu.sync_copy(cum_hbm, cur)

            def init(i, _):
                parent[pl.ds(i * 16, 16)] = lax.iota(jnp.int32, 16) + i * 16
                pers[pl.ds(i * 16, 16)] = jnp.full((16,), -1.0, jnp.float32)
                return 0

            lax.fori_loop(0, N // 16, init, 0)

            def setb(w, _):
                _sstore(bnd, w, _locate(w * TGT))
                return 0

            lax.fori_loop(0, NSUB + 1, setb, 0)

            def scan_window(w, carry):
                wlo = _sload(bnd, w)
                whi = _sload(bnd, w + 1)
                wcnt = _cumval(whi) - _cumval(wlo)
                pltpu.sync_copy(swin_hbm.at[pl.ds(w * CAPW, CAPW)], win)

                def blk(b, c2):
                    ev = win[pl.ds(b * 16, 16)]
                    ov = jnp.bitwise_and(ev, 4095)
                    kv = lax.shift_right_logical(ev, 12)

                    def pf_cond(c3):
                        return c3[1]

                    def pf_body(c3):
                        r, _ = c3
                        pr = plsc.load_gather(parent, [r])
                        moved = plsc.all_reduce_population_count(pr != r)[0]
                        return (pr, moved != 0)

                    rts, _ = lax.while_loop(pf_cond, pf_body,
                                            (ov, jnp.bool_(True)))

                    cur_key, rk = c2
                    for lane in range(16):
                        valid = b * 16 + lane < wcnt
                        k = kv[lane]
                        o = ov[lane]
                        rk = jnp.where(jnp.logical_and(valid, k != cur_key),
                                       k, rk)
                        cur_key = jnp.where(valid, k, cur_key)

                        def find_cond(r):
                            return _sload(parent, r) != r

                        ro = lax.while_loop(find_cond,
                                            lambda r: _sload(parent, r),
                                            rts[lane])
                        dying = jnp.maximum(rk, ro)
                        main = jnp.minimum(rk, ro)
                        do_union = jnp.logical_and(valid, ro != rk)

                        @pl.when(valid)
                        def _():
                            _sstore(parent, o, ro)

                        @pl.when(do_union)
                        def _():
                            _sstore(parent, dying, main)

                            @pl.when(dying != k)
                            def _():
                                _sstore(pers, dying,
                                        _sload(fs_v, dying) - _sload(fs_v, k))

                        rk = jnp.where(do_union, main, rk)
                    return (cur_key, rk)

                return lax.fori_loop(0, (wcnt + 15) // 16, blk, carry)

            lax.fori_loop(0, NSUB, scan_window,
                          (jnp.int32(-1), jnp.int32(-1)))

            fmin = _sload(fs_v, N - 1)

            def fin(i, carry):
                sumsq, kcnt = carry
                base = i * 16
                rvec = lax.iota(jnp.int32, 16) + base
                par = parent[pl.ds(base, 16)]
                death = pers[pl.ds(base, 16)]
                fsv = fs_v[pl.ds(base, 16)]
                is_root = par == rvec
                valid = jnp.logical_or(is_root, death >= 0.0)
                val = jnp.where(is_root, fsv - fmin, death)
                val = jnp.where(valid, val, jnp.float32(-1.0))
                pers[pl.ds(base, 16)] = val
                sumsq = sumsq + jnp.sum(jnp.where(valid, val * val, 0.0))
                kcnt = kcnt + jnp.sum(jnp.where(valid, 1, 0).astype(jnp.int32))
                return (sumsq, kcnt)

            sumsq, kcnt = lax.fori_loop(0, N // 16, fin,
                                        (jnp.float32(0.0), jnp.int32(0)))

            def top_j(j, carry):
                loss = carry

                def mx(i, m):
                    return jnp.maximum(m, jnp.max(pers[pl.ds(i * 16, 16)]))

                m = lax.fori_loop(0, N // 16, mx, jnp.float32(-3.0))

                def floc(i, st):
                    found, pos = st
                    chunk = pers[pl.ds(i * 16, 16)]
                    hit = jnp.logical_and(jnp.logical_not(found),
                                          jnp.max(chunk) >= m)
                    lane = jnp.max(plsc.all_reduce_ffs(chunk == m))
                    return (jnp.logical_or(found, hit),
                            jnp.where(hit, i * 16 + lane, pos))

                _, pos = lax.fori_loop(0, N // 16, floc,
                                       (jnp.bool_(False), jnp.int32(0)))
                _sstore(pers, pos, jnp.float32(-3.0))
                return loss + jnp.where(j < jnp.minimum(kcnt, DESTNUM),
                                        1.0 - 2.0 * m, 0.0)

            loss = lax.fori_loop(0, DESTNUM, top_j, sumsq)
            obuf[...] = jnp.full((16,), 0.0, jnp.float32) + loss
            pltpu.sync_copy(obuf, out_hbm)


def _persistence_loss(codes, fsorted):
    mesh = plsc.VectorSubcoreMesh(core_axis_name="c", subcore_axis_name="s")
    kfn = pl.kernel(
        _k4_body,
        mesh=mesh,
        compiler_params=_SC_PARAMS,
        out_type=[jax.ShapeDtypeStruct((16,), jnp.float32),
                  jax.ShapeDtypeStruct((NSUB, N), jnp.int32),
                  jax.ShapeDtypeStruct((N,), jnp.int32),
                  jax.ShapeDtypeStruct((NSUB * CAPW,), jnp.int32)],
        scratch_types=[pltpu.VMEM((N,), jnp.float32),
                       pltpu.VMEM((N,), jnp.int32),
                       pltpu.VMEM((N,), jnp.float32),
                       pltpu.VMEM((N,), jnp.int32),
                       pltpu.VMEM((N,), jnp.int32),
                       pltpu.VMEM((CAPW,), jnp.int32),
                       pltpu.VMEM((CHUNK,), jnp.int32),
                       pltpu.VMEM((32,), jnp.int32),
                       pltpu.VMEM((16,), jnp.float32),
                       pltpu.SemaphoreType.DMA],
    )
    return kfn(codes, fsorted)[0]


def kernel(x):
    f, nbrs = _density_topk(x)
    rank = _ranks(f)
    codes, fsorted = _edge_codes(nbrs.reshape(E), rank, f)
    out = _persistence_loss(codes, fsorted)
    return out[0]
